# baseline (speedup 1.0000x reference)
import jax
import jax.numpy as jnp
from jax import lax
from jax.experimental import pallas as pl
from jax.experimental.pallas import tpu as pltpu
from jax.experimental.pallas import tpu_sc as plsc

NC, NS = 2, 16
NW = NC * NS
CHUNK = 128
D = 64
VOC = 1000000
NBLK = VOC // CHUNK
TAIL0 = NBLK * CHUNK
PAIRS = VOC // 2


def _relayout_body(tt_hbm, tail_hbm, tp_hbm, i0, i1, c0, c1, tl_v, gi0, gi1, so0, so1):
    ins = (i0, i1)
    cmp = (c0, c1)
    isems = (gi0, gi1)
    osems = (so0, so1)

    wid = lax.axis_index("s") * NC + lax.axis_index("c")
    nit = (NBLK + NW - 1) // NW

    def fire_in(i, p):
        blk = wid + NW * i
        pltpu.async_copy(tt_hbm.at[:, pl.ds(blk * CHUNK, CHUNK)], ins[p],
                         isems[p])

    def drain_in(p):
        pltpu.make_async_copy(tt_hbm.at[:, pl.ds(0, CHUNK)], ins[p],
                              isems[p]).wait()

    def fire_out(i, p):
        blk = wid + NW * i
        pltpu.async_copy(cmp[p], tp_hbm.at[pl.ds(blk * (CHUNK // 2),
                                                 CHUNK // 2)], osems[p])

    def drain_out(p):
        pltpu.make_async_copy(cmp[p], tp_hbm.at[pl.ds(0, CHUNK // 2)],
                              osems[p]).wait()

    def transpose(p):
        rows16 = lax.iota(jnp.int32, 16)

        @pl.loop(0, CHUNK)
        def _(c):
            r = c // 2
            h = (c % 2) * D
            cols16 = jnp.full((16,), c, jnp.int32)
            for k in range(4):
                v = plsc.load_gather(ins[p], [rows16 + k * 16, cols16])
                cmp[p][r, pl.ds(h + k * 16, 16)] = v

    @pl.when(wid == 0)
    def _():
        pltpu.sync_copy(tail_hbm, tl_v)
        pltpu.sync_copy(tl_v, tp_hbm.at[pl.ds(PAIRS - (VOC - TAIL0) // 2,
                                              (VOC - TAIL0) // 2)])

    @pl.when(wid < NBLK)
    def _():
        fire_in(0, 0)

    @pl.loop(0, nit, step=2)
    def _(ii):
        for p in range(2):
            i = ii + p

            @pl.when(wid + NW * i < NBLK)
            def _():
                @pl.when(wid + NW * (i + 1) < NBLK)
                def _():
                    fire_in(i + 1, 1 - p)

                drain_in(p)

                @pl.when(i >= 2)
                def _():
                    drain_out(p)

                transpose(p)
                fire_out(i, p)

    for p in range(2):
        @pl.when(wid + NW * (nit - 2 + p) < NBLK)
        def _():
            drain_out(p)


def _gather_body(table_hbm, idx_hbm, out_hbm,
          idx_v, p0, p1, r0, r1, c0, c1, g0, g1, s0, s1):
    pidx = (p0, p1)
    rows = (r0, r1)
    cmp = (c0, c1)
    gsems = (g0, g1)
    ssems = (s0, s1)

    wid = lax.axis_index("s") * NC + lax.axis_index("c")
    nch = idx_hbm.shape[1]
    base = wid * (nch * CHUNK)

    pltpu.sync_copy(idx_hbm.at[wid], idx_v)

    def fire_gather(j, p):
        @pl.loop(0, CHUNK // 16)
        def _(g):
            v = idx_v[j, pl.ds(g * 16, 16)]
            pidx[p][pl.ds(g * 16, 16)] = v >> 1
        pltpu.async_copy(table_hbm.at[pidx[p]], rows[p], gsems[p])

    def drain_gather(p):
        pltpu.make_async_copy(table_hbm.at[pl.ds(0, CHUNK)], rows[p],
                              gsems[p]).wait()

    def select(j, p):
        @pl.loop(0, CHUNK // 16)
        def _(g):
            v16 = idx_v[j, pl.ds(g * 16, 16)]
            par16 = (v16 & 1) * 64
            for l in range(16):
                i = g * 16 + l
                col0 = par16[l]
                for k in range(4):
                    cmp[p][i, pl.ds(k * 16, 16)] = \
                        rows[p][i, pl.ds(col0 + k * 16, 16)]

    def fire_store(j, p):
        pltpu.async_copy(cmp[p], out_hbm.at[pl.ds(base + j * CHUNK, CHUNK)],
                         ssems[p])

    def drain_store(p):
        pltpu.make_async_copy(cmp[p], out_hbm.at[pl.ds(base, CHUNK)],
                              ssems[p]).wait()

    fire_gather(0, 0)

    @pl.loop(0, nch, step=2)
    def _(jj):
        for p in range(2):
            j = jj + p

            @pl.when(j + 1 < nch)
            def _():
                fire_gather(j + 1, 1 - p)

            drain_gather(p)

            @pl.when(j >= 2)
            def _():
                drain_store(p)

            select(j, p)
            fire_store(j, p)

    drain_store(0)
    drain_store(1)


def kernel(token_ids, table):
    bt, s = token_ids.shape
    b = bt * s
    nch = b // (NW * CHUNK)
    idx = token_ids.reshape(NW, nch, CHUNK).astype(jnp.int32)
    table_t = table.T
    tail = table[TAIL0:, :].reshape((VOC - TAIL0) // 2, 2 * D)

    mesh = plsc.VectorSubcoreMesh(core_axis_name="c", subcore_axis_name="s")
    cparams = pltpu.CompilerParams(use_tc_tiling_on_sc=True,
                                   needs_layout_passes=False)

    relayout = pl.kernel(
        _relayout_body,
        out_type=jax.ShapeDtypeStruct((PAIRS, 2 * D), table.dtype),
        mesh=mesh,
        scratch_types=(
            [pltpu.VMEM((D, CHUNK), jnp.float32) for _ in range(2)]
            + [pltpu.VMEM((CHUNK // 2, 2 * D), jnp.float32) for _ in range(2)]
            + [pltpu.VMEM(((VOC - TAIL0) // 2, 2 * D), jnp.float32)]
            + [pltpu.SemaphoreType.DMA for _ in range(4)]
        ),
        compiler_params=pltpu.CompilerParams(use_tc_tiling_on_sc=True,
                                             needs_layout_passes=False),
    )
    table_pairs = relayout(table_t, tail)

    gather = pl.kernel(
        _gather_body,
        out_type=jax.ShapeDtypeStruct((b, D), table.dtype),
        mesh=mesh,
        scratch_types=(
            [pltpu.VMEM((nch, CHUNK), jnp.int32)]
            + [pltpu.VMEM((CHUNK,), jnp.int32) for _ in range(2)]
            + [pltpu.VMEM((CHUNK, 2 * D), jnp.float32) for _ in range(2)]
            + [pltpu.VMEM((CHUNK, D), jnp.float32) for _ in range(2)]
            + [pltpu.SemaphoreType.DMA for _ in range(4)]
        ),
        compiler_params=cparams,
    )
    out = gather(table_pairs, idx)
    return out.reshape(bt, s, D)


# speedup vs baseline: 1.7744x; 1.7744x over previous
import jax
import jax.numpy as jnp
from jax import lax
from jax.experimental import pallas as pl
from jax.experimental.pallas import tpu as pltpu
from jax.experimental.pallas import tpu_sc as plsc

NC, NS = 2, 16
NW = NC * NS
CHUNK = 128
D = 64


def _body(table_hbm, idx_hbm, out_hbm,
          idx_v, p0, p1, r0, r1, c0, c1, g0, g1, s0, s1):
    pidx = (p0, p1)
    rows = (r0, r1)
    cmp = (c0, c1)
    gsems = (g0, g1)
    ssems = (s0, s1)

    wid = lax.axis_index("s") * NC + lax.axis_index("c")
    nch = idx_hbm.shape[1]
    base = wid * (nch * CHUNK)

    pltpu.sync_copy(idx_hbm.at[wid], idx_v)

    def fire_gather(j, p):
        @pl.loop(0, CHUNK // 16)
        def _(g):
            v = idx_v[j, pl.ds(g * 16, 16)]
            pidx[p][pl.ds(g * 16, 16)] = v >> 1
        pltpu.async_copy(table_hbm.at[pidx[p]], rows[p], gsems[p])

    def drain_gather(p):
        pltpu.make_async_copy(table_hbm.at[pl.ds(0, CHUNK)], rows[p],
                              gsems[p]).wait()

    def select(j, p):
        @pl.loop(0, CHUNK // 16)
        def _(g):
            v16 = idx_v[j, pl.ds(g * 16, 16)]
            par16 = (v16 & 1) * 64
            for l in range(16):
                i = g * 16 + l
                col0 = par16[l]
                for k in range(4):
                    cmp[p][i, pl.ds(k * 16, 16)] = \
                        rows[p][i, pl.ds(col0 + k * 16, 16)]

    def fire_store(j, p):
        pltpu.async_copy(cmp[p], out_hbm.at[pl.ds(base + j * CHUNK, CHUNK)],
                         ssems[p])

    def drain_store(p):
        pltpu.make_async_copy(cmp[p], out_hbm.at[pl.ds(base, CHUNK)],
                              ssems[p]).wait()

    fire_gather(0, 0)

    @pl.loop(0, nch, step=2)
    def _(jj):
        for p in range(2):
            j = jj + p

            @pl.when(j + 1 < nch)
            def _():
                fire_gather(j + 1, 1 - p)

            drain_gather(p)

            @pl.when(j >= 2)
            def _():
                drain_store(p)

            select(j, p)
            fire_store(j, p)

    drain_store(0)
    drain_store(1)


def kernel(token_ids, table):
    bt, s = token_ids.shape
    b = bt * s
    nch = b // (NW * CHUNK)
    idx = token_ids.reshape(NW, nch, CHUNK).astype(jnp.int32)
    table_pairs = table.reshape(table.shape[0] // 2, 2 * D)

    mesh = plsc.VectorSubcoreMesh(core_axis_name="c", subcore_axis_name="s")
    run = pl.kernel(
        _body,
        out_type=jax.ShapeDtypeStruct((b, D), table.dtype),
        mesh=mesh,
        scratch_types=(
            [pltpu.VMEM((nch, CHUNK), jnp.int32)]
            + [pltpu.VMEM((CHUNK,), jnp.int32) for _ in range(2)]
            + [pltpu.VMEM((CHUNK, 2 * D), jnp.float32) for _ in range(2)]
            + [pltpu.VMEM((CHUNK, D), jnp.float32) for _ in range(2)]
            + [pltpu.SemaphoreType.DMA for _ in range(4)]
        ),
        compiler_params=pltpu.CompilerParams(use_tc_tiling_on_sc=True),
    )
    out = run(table_pairs, idx)
    return out.reshape(bt, s, D)
